# baseline (speedup 1.0000x reference)
import functools

import jax
import jax.numpy as jnp
from jax import lax
from jax.experimental import pallas as pl
from jax.experimental.pallas import tpu as pltpu
from jax.experimental.pallas import tpu_sc as plsc


def _build_kernel(F, V, D, B):
    info = plsc.get_sparse_core_info()
    NC, NS = info.num_cores, info.num_subcores
    NW = NC * NS
    rows_per_w = B // NW
    CB = 32
    nchunk = rows_per_w // CB
    PC = CB * F
    NVEC = PC // 16
    GCH = 104
    NG = PC // GCH
    KD = D // 16
    inv_n = 1.0 / float(F * D)

    mesh = plsc.VectorSubcoreMesh(core_axis_name="c", subcore_axis_name="s")

    @functools.partial(
        pl.kernel,
        out_type=jax.ShapeDtypeStruct((B * F, D), jnp.float32),
        mesh=mesh,
        compiler_params=pltpu.CompilerParams(use_tc_tiling_on_sc=False),
        scratch_types=[
            pltpu.VMEM((PC,), jnp.int32),
            pltpu.VMEM((PC,), jnp.int32),
            pltpu.VMEM((PC, D), jnp.float32),
            pltpu.VMEM((F * D,), jnp.float32),
            pltpu.VMEM((F * D,), jnp.float32),
            pltpu.SemaphoreType.DMA,
        ],
    )
    def body(x_hbm, tab_hbm, gamma_hbm, beta_hbm, out_hbm,
             raw_v, gidx_v, g_v, gam_v, bet_v, sem):
        wid = lax.axis_index("s") * NC + lax.axis_index("c")
        pltpu.sync_copy(gamma_hbm, gam_v)
        pltpu.sync_copy(beta_hbm, bet_v)
        lane = lax.iota(jnp.int32, 16)
        perms = [lane ^ sh for sh in (8, 4, 2, 1)]

        gdn = lax.GatherDimensionNumbers(
            offset_dims=(), collapsed_slice_dims=(0,), start_index_map=(0,))

        def lane_total(v):
            for p in perms:
                v = v + lax.gather(
                    v, p[:, None], dimension_numbers=gdn, slice_sizes=(1,),
                    mode=lax.GatherScatterMode.PROMISE_IN_BOUNDS)
            return v

        def chunk_body(c, carry):
            base_p = (wid * nchunk + c) * PC
            pltpu.sync_copy(x_hbm.at[pl.ds(base_p, PC)], raw_v)

            def idx_body(i, carry2):
                r = raw_v[pl.ds(i * 16, 16)]
                f = lax.rem(i * 16 + lane, F)
                r = jnp.minimum(jnp.maximum(r, 0), V - 1)
                gidx_v[pl.ds(i * 16, 16)] = r + f * V
                return carry2

            lax.fori_loop(0, NVEC, idx_body, 0)

            copies = []
            for gch in range(NG):
                copies.append(pltpu.async_copy(
                    tab_hbm.at[gidx_v.at[pl.ds(gch * GCH, GCH)]],
                    g_v.at[pl.ds(gch * GCH, GCH)], sem))
            for cp in copies:
                cp.wait()

            def row_body(b, carry2):
                p0 = b * F

                def stat_body(f, sq):
                    s, q = sq
                    for k in range(KD):
                        v = g_v[p0 + f, pl.ds(k * 16, 16)]
                        s = s + v
                        q = q + v * v
                    return (s, q)

                zeros = jnp.zeros((16,), jnp.float32)
                s, q = lax.fori_loop(0, F, stat_body, (zeros, zeros))
                mean = lane_total(s) * inv_n
                var = lane_total(q) * inv_n - mean * mean
                av = var + 1e-5
                ii = lax.bitcast_convert_type(av, jnp.int32)
                ii = 0x5F3759DF - lax.shift_right_arithmetic(ii, 1)
                y = lax.bitcast_convert_type(ii, jnp.float32)
                y = y * (1.5 - 0.5 * av * y * y)
                y = y * (1.5 - 0.5 * av * y * y)
                y = y * (1.5 - 0.5 * av * y * y)
                c1 = y
                c0 = -mean * y

                def norm_body(f, carry3):
                    for k in range(KD):
                        sl = pl.ds(k * 16, 16)
                        gsl = pl.ds(f * D + k * 16, 16)
                        v = g_v[p0 + f, sl]
                        t = v * c1 + c0
                        g_v[p0 + f, sl] = t * gam_v[gsl] + bet_v[gsl]
                    return carry3

                lax.fori_loop(0, F, norm_body, 0)
                return carry2

            lax.fori_loop(0, CB, row_body, 0)
            pltpu.sync_copy(g_v, out_hbm.at[pl.ds(base_p, PC)])
            return carry

        lax.fori_loop(0, nchunk, chunk_body, 0)

    return body


def kernel(x_cat, tables, gamma, beta):
    B, F = x_cat.shape
    _, V, D = tables.shape
    x_flat = x_cat.reshape(B * F)
    tab_flat = tables.reshape(F * V, D)
    out = _build_kernel(F, V, D, B)(x_flat, tab_flat, gamma, beta)
    return out.reshape(B, F * D)
